# baseline (speedup 1.0000x reference)
import functools

import jax
import jax.numpy as jnp
from jax import lax
from jax.experimental import pallas as pl
from jax.experimental.pallas import tpu as pltpu
from jax.experimental.pallas import tpu_sc as plsc

N = 10000
E = 320000
H = 128
G = 128
NC, NS, LANES = 2, 16, 16
NW = NC * NS
C = 128
CHUNKS = 80
CHUNKS_ST = CHUNKS + 8
C2 = 2 * C
M_ST = 24
M_STAGES = ((0, 0, 20), (16, 4, 20))
E_PAD = NW * CHUNKS * C
ACC_ROWS = 10240
ZROWS = ACC_ROWS // NS
DUMMY = N

@functools.cache
def _get_mesh():
    return plsc.VectorSubcoreMesh(
        core_axis_name="c", subcore_axis_name="s", num_cores=NC, num_subcores=NS)


_HIGH = lax.Precision.HIGHEST


def _fill(buf, rows, cols, val):
    @pl.loop(0, rows)
    def _(r):
        @pl.loop(0, cols, step=LANES)
        def _(cc):
            buf[r, pl.ds(cc, LANES)] = jnp.full((LANES,), val, buf.dtype)


def _deg_body(dst_hbm, out_hbm, didx, buf, acc, sem):
    cid = lax.axis_index("c")
    sid = lax.axis_index("s")
    wid = cid * NS + sid
    pltpu.sync_copy(dst_hbm.at[wid], didx)
    _fill(buf, C, LANES, 0.0)
    base = sid * ZROWS
    for k in range(ZROWS // C):
        pltpu.sync_copy(buf, acc.at[pl.ds(base + k * C, C)])
    _fill(buf, C, LANES, 1.0)
    plsc.subcore_barrier()

    @pl.loop(0, CHUNKS)
    def _(j):
        pltpu.sync_copy(buf, acc.at[didx.at[j]], add=True)

    plsc.subcore_barrier()
    pltpu.sync_copy(acc.at[pl.ds(base, ZROWS)],
                    out_hbm.at[cid, pl.ds(base, ZROWS)])


def _mp_body(xs_hbm, src_hbm, dst_hbm, out_hbm, sidx, didx, gbuf_a, acc,
             sem_a):
    cid = lax.axis_index("c")
    sid = lax.axis_index("s")
    wid = cid * NS + sid
    _fill(gbuf_a, C, H, 0.0)
    base = sid * ZROWS
    for k in range(ZROWS // C):
        pltpu.sync_copy(gbuf_a, acc.at[pl.ds(base + k * C, C)])
    plsc.subcore_barrier()

    pltpu.sync_copy(src_hbm.at[wid], sidx)
    pltpu.sync_copy(dst_hbm.at[wid], didx)

    @pl.loop(0, CHUNKS)
    def _(j):
        pltpu.async_copy(xs_hbm.at[sidx.at[j]], gbuf_a, sem_a).wait()
        pltpu.sync_copy(gbuf_a, acc.at[didx.at[j]], add=True)

    plsc.subcore_barrier()
    pltpu.sync_copy(acc.at[pl.ds(base, ZROWS)],
                    out_hbm.at[cid, pl.ds(base, ZROWS)])


def _xw_body(x_ref, w_ref, xw_ref):
    xw_ref[...] = jnp.dot(x_ref[...], w_ref[...],
                          preferred_element_type=jnp.float32, precision=_HIGH)


def _prep_body(deg2_ref, xw_ref, xs_ref, dinv_ref):
    d0 = deg2_ref[0]
    d1 = deg2_ref[1]
    deg = d0[:N, 0:1] + d1[:N, 0:1] + 1.0
    dinv = lax.rsqrt(deg)
    dinv_ref[...] = dinv
    xs_ref[...] = xw_ref[...] * dinv


def _bn_relu(conv, g, be):
    m = jnp.mean(conv, axis=0, keepdims=True)
    cc = conv - m
    v = jnp.mean(cc * cc, axis=0, keepdims=True)
    return jnp.maximum(cc * lax.rsqrt(v + 1e-5) * g + be, 0.0)


def _conv_body(p_ref, xs_ref, dinv_ref, b_ref, conv_ref):
    conv_ref[...] = ((p_ref[0][:N] + p_ref[1][:N] + xs_ref[...])
                     * dinv_ref[...] + b_ref[...])


def _bn1_body(conv_ref, g_ref, be_ref, dinv_ref, w_ref, h_ref, xsn_ref):
    h = _bn_relu(conv_ref[...], g_ref[...], be_ref[...])
    h_ref[...] = h
    xsn_ref[...] = jnp.dot(h, w_ref[...], preferred_element_type=jnp.float32,
                           precision=_HIGH) * dinv_ref[...]


def _bn2_body(conv_ref, g_ref, be_ref, res_ref, dinv_ref, w_ref,
              h_ref, xsn_ref):
    h = _bn_relu(conv_ref[...], g_ref[...], be_ref[...]) + res_ref[...]
    h_ref[...] = h
    xsn_ref[...] = jnp.dot(h, w_ref[...], preferred_element_type=jnp.float32,
                           precision=_HIGH) * dinv_ref[...]


def _bn3_body(conv_ref, g_ref, be_ref, res_ref, h_ref):
    h_ref[...] = _bn_relu(conv_ref[...], g_ref[...], be_ref[...]) + res_ref[...]


def _final_body(h_ref, batch_ref, mw1a_ref, mw1b_ref, mb1_ref, mw2_ref,
                mb2_ref, out_ref):
    h = h_ref[...]
    gids = lax.broadcasted_iota(jnp.int32, (G, N), 0)
    mask = (gids == batch_ref[...]).astype(jnp.float32)
    xs_pool = jnp.dot(mask, h, preferred_element_type=jnp.float32,
                      precision=_HIGH)
    cnt = jnp.sum(mask, axis=1, keepdims=True)
    xm = xs_pool / jnp.maximum(cnt, 1.0)
    z = jnp.dot(xs_pool, mw1a_ref[...], preferred_element_type=jnp.float32,
                precision=_HIGH)
    z = z + jnp.dot(xm, mw1b_ref[...], preferred_element_type=jnp.float32,
                    precision=_HIGH)
    z = jnp.maximum(z + mb1_ref[...], 0.0)
    out_ref[...] = jnp.dot(z, mw2_ref[...], preferred_element_type=jnp.float32,
                           precision=_HIGH) + mb2_ref[...]


def _sds(shape):
    return jax.ShapeDtypeStruct(shape, jnp.float32)


@functools.cache
def _get_deg_kernel():
    return pl.kernel(
        _deg_body,
        out_type=jax.ShapeDtypeStruct((NC, ACC_ROWS, LANES), jnp.float32),
        mesh=_get_mesh(),
        scratch_types=[
            pltpu.VMEM((CHUNKS_ST, C), jnp.int32),
            pltpu.VMEM((C, LANES), jnp.float32),
            pltpu.VMEM_SHARED((ACC_ROWS, LANES), jnp.float32),
            pltpu.SemaphoreType.DMA,
        ],
    )


@functools.cache
def _get_mp_kernel():
    return pl.kernel(
        _mp_body,
        out_type=jax.ShapeDtypeStruct((NC, ACC_ROWS, H), jnp.float32),
        mesh=_get_mesh(),
        scratch_types=[
            pltpu.VMEM((CHUNKS_ST, C), jnp.int32),
            pltpu.VMEM((CHUNKS_ST, C), jnp.int32),
            pltpu.VMEM((C, H), jnp.float32),
            pltpu.VMEM_SHARED((ACC_ROWS, H), jnp.float32),
            pltpu.SemaphoreType.DMA,
        ],
    )


def kernel(x, edge_index, batch, W1, b1, g1, be1, W2, b2, g2, be2,
           W3, b3, g3, be3, MW1, Mb1, MW2, Mb2):
    pad = E_PAD - E
    srcp = jnp.concatenate(
        [edge_index[0], jnp.zeros((pad,), jnp.int32)]).reshape(NW, CHUNKS, C)
    dstp = jnp.concatenate(
        [edge_index[1], jnp.full((pad,), DUMMY, jnp.int32)]).reshape(NW, CHUNKS, C)
    ext = CHUNKS_ST - CHUNKS
    srcp = jnp.concatenate([srcp, jnp.zeros((NW, ext, C), jnp.int32)], axis=1)
    dstp = jnp.concatenate(
        [dstp, jnp.full((NW, ext, C), DUMMY, jnp.int32)], axis=1)
    batch2 = batch.reshape(1, N)
    b1r, g1r, be1r = b1.reshape(1, H), g1.reshape(1, H), be1.reshape(1, H)
    b2r, g2r, be2r = b2.reshape(1, H), g2.reshape(1, H), be2.reshape(1, H)
    b3r, g3r, be3r = b3.reshape(1, H), g3.reshape(1, H), be3.reshape(1, H)
    mw1a, mw1b = MW1[:H], MW1[H:]
    mb1 = Mb1.reshape(1, H)
    mb2 = Mb2.reshape(1, 1)

    deg2 = _get_deg_kernel()(dstp)
    xw1 = pl.pallas_call(_xw_body, out_shape=_sds((N, H)))(x, W1)
    xs1, dinv = pl.pallas_call(
        _prep_body, out_shape=(_sds((N, H)), _sds((N, 1))))(deg2, xw1)

    p1 = _get_mp_kernel()(xs1, srcp, dstp)
    conv1 = pl.pallas_call(_conv_body, out_shape=_sds((N, H)))(
        p1, xs1, dinv, b1r)
    h1, xs2 = pl.pallas_call(
        _bn1_body, out_shape=(_sds((N, H)), _sds((N, H))))(
            conv1, g1r, be1r, dinv, W2)

    p2 = _get_mp_kernel()(xs2, srcp, dstp)
    conv2 = pl.pallas_call(_conv_body, out_shape=_sds((N, H)))(
        p2, xs2, dinv, b2r)
    h2, xs3 = pl.pallas_call(
        _bn2_body, out_shape=(_sds((N, H)), _sds((N, H))))(
            conv2, g2r, be2r, h1, dinv, W3)

    p3 = _get_mp_kernel()(xs3, srcp, dstp)
    conv3 = pl.pallas_call(_conv_body, out_shape=_sds((N, H)))(
        p3, xs3, dinv, b3r)
    h3 = pl.pallas_call(_bn3_body, out_shape=_sds((N, H)))(
        conv3, g3r, be3r, h2)
    out = pl.pallas_call(_final_body, out_shape=_sds((G, 1)))(
        h3, batch2, mw1a, mw1b, mb1, MW2, mb2)
    return out


# speedup vs baseline: 1.0014x; 1.0014x over previous
import functools

import jax
import jax.numpy as jnp
from jax import lax
from jax.experimental import pallas as pl
from jax.experimental.pallas import tpu as pltpu
from jax.experimental.pallas import tpu_sc as plsc

N = 10000
E = 320000
H = 128
G = 128
NC, NS, LANES = 2, 16, 16
NW = NC * NS
C = 128
CHUNKS = 80
CHUNKS_ST = CHUNKS + 8
C2 = 2 * C
M_ST = 24
M_STAGES = ((0, 0, 20), (16, 4, 20))
E_PAD = NW * CHUNKS * C
ACC_ROWS = 10240
ZROWS = ACC_ROWS // NS
DUMMY = N

@functools.cache
def _get_mesh():
    return plsc.VectorSubcoreMesh(
        core_axis_name="c", subcore_axis_name="s", num_cores=NC, num_subcores=NS)


_HIGH = lax.Precision.HIGHEST


def _fill(buf, rows, cols, val):
    @pl.loop(0, rows)
    def _(r):
        @pl.loop(0, cols, step=LANES)
        def _(cc):
            buf[r, pl.ds(cc, LANES)] = jnp.full((LANES,), val, buf.dtype)


def _deg_body(dst_hbm, out_hbm, didx, buf, acc, sem):
    cid = lax.axis_index("c")
    sid = lax.axis_index("s")
    wid = cid * NS + sid
    pltpu.sync_copy(dst_hbm.at[wid], didx)
    _fill(buf, C, LANES, 0.0)
    base = sid * ZROWS
    for k in range(ZROWS // C):
        pltpu.sync_copy(buf, acc.at[pl.ds(base + k * C, C)])
    _fill(buf, C, LANES, 1.0)
    plsc.subcore_barrier()

    @pl.loop(0, CHUNKS)
    def _(j):
        pltpu.sync_copy(buf, acc.at[didx.at[j]], add=True)

    plsc.subcore_barrier()
    pltpu.sync_copy(acc.at[pl.ds(base, ZROWS)],
                    out_hbm.at[cid, pl.ds(base, ZROWS)])


def _mp_body(xs_hbm, src_hbm, dst_hbm, out_hbm, sidx, didx, gbuf_a, acc,
             sem_a):
    cid = lax.axis_index("c")
    sid = lax.axis_index("s")
    wid = cid * NS + sid
    _fill(gbuf_a, C, H, 0.0)
    base = sid * ZROWS
    for k in range(ZROWS // C):
        pltpu.sync_copy(gbuf_a, acc.at[pl.ds(base + k * C, C)])
    plsc.subcore_barrier()

    pltpu.sync_copy(src_hbm.at[wid], sidx)
    pltpu.sync_copy(dst_hbm.at[wid], didx)

    @pl.loop(0, CHUNKS)
    def _(j):
        pltpu.async_copy(xs_hbm.at[sidx.at[j]], gbuf_a, sem_a).wait()
        pltpu.sync_copy(gbuf_a, acc.at[didx.at[j]], add=True)

    plsc.subcore_barrier()
    pltpu.sync_copy(acc.at[pl.ds(base, ZROWS)],
                    out_hbm.at[cid, pl.ds(base, ZROWS)])


def _xw_body(x_ref, w_ref, xw_ref):
    xw_ref[...] = jnp.dot(x_ref[...], w_ref[...],
                          preferred_element_type=jnp.float32, precision=_HIGH)


def _prep_body(deg2_ref, xw_ref, xs_ref, dinv_ref):
    d0 = deg2_ref[0]
    d1 = deg2_ref[1]
    deg = d0[:N, 0:1] + d1[:N, 0:1] + 1.0
    dinv = lax.rsqrt(deg)
    dinv_ref[...] = dinv
    xs_ref[...] = xw_ref[...] * dinv


def _bn_relu(conv, g, be):
    m = jnp.mean(conv, axis=0, keepdims=True)
    cc = conv - m
    v = jnp.mean(cc * cc, axis=0, keepdims=True)
    return jnp.maximum(cc * lax.rsqrt(v + 1e-5) * g + be, 0.0)


def _conv_body(p_ref, xs_ref, dinv_ref, b_ref, conv_ref):
    conv_ref[...] = ((p_ref[0][:N] + p_ref[1][:N] + xs_ref[...])
                     * dinv_ref[...] + b_ref[...])


def _bn1_body(conv_ref, g_ref, be_ref, dinv_ref, w_ref, h_ref, xsn_ref):
    h = _bn_relu(conv_ref[...], g_ref[...], be_ref[...])
    h_ref[...] = h
    xsn_ref[...] = jnp.dot(h, w_ref[...], preferred_element_type=jnp.float32,
                           precision=_HIGH) * dinv_ref[...]


def _bn2_body(conv_ref, g_ref, be_ref, res_ref, dinv_ref, w_ref,
              h_ref, xsn_ref):
    h = _bn_relu(conv_ref[...], g_ref[...], be_ref[...]) + res_ref[...]
    h_ref[...] = h
    xsn_ref[...] = jnp.dot(h, w_ref[...], preferred_element_type=jnp.float32,
                           precision=_HIGH) * dinv_ref[...]


def _bn3_body(conv_ref, g_ref, be_ref, res_ref, h_ref):
    h_ref[...] = _bn_relu(conv_ref[...], g_ref[...], be_ref[...]) + res_ref[...]


def _final_body(h_ref, batch_ref, mw1a_ref, mw1b_ref, mb1_ref, mw2_ref,
                mb2_ref, out_ref):
    h = h_ref[...]
    gids = lax.broadcasted_iota(jnp.int32, (G, N), 0)
    mask = (gids == batch_ref[...]).astype(jnp.float32)
    xs_pool = jnp.dot(mask, h, preferred_element_type=jnp.float32,
                      precision=_HIGH)
    cnt = jnp.sum(mask, axis=1, keepdims=True)
    xm = xs_pool / jnp.maximum(cnt, 1.0)
    z = jnp.dot(xs_pool, mw1a_ref[...], preferred_element_type=jnp.float32,
                precision=_HIGH)
    z = z + jnp.dot(xm, mw1b_ref[...], preferred_element_type=jnp.float32,
                    precision=_HIGH)
    z = jnp.maximum(z + mb1_ref[...], 0.0)
    out_ref[...] = jnp.dot(z, mw2_ref[...], preferred_element_type=jnp.float32,
                           precision=_HIGH) + mb2_ref[...]


def _sds(shape):
    return jax.ShapeDtypeStruct(shape, jnp.float32)


@functools.cache
def _get_deg_kernel():
    return pl.kernel(
        _deg_body,
        out_type=jax.ShapeDtypeStruct((NC, ACC_ROWS, LANES), jnp.float32),
        mesh=_get_mesh(),
        scratch_types=[
            pltpu.VMEM((CHUNKS_ST, C), jnp.int32),
            pltpu.VMEM((C, LANES), jnp.float32),
            pltpu.VMEM_SHARED((ACC_ROWS, LANES), jnp.float32),
            pltpu.SemaphoreType.DMA,
        ],
    )


@functools.cache
def _get_mp_kernel():
    return pl.kernel(
        _mp_body,
        out_type=jax.ShapeDtypeStruct((NC, ACC_ROWS, H), jnp.float32),
        mesh=_get_mesh(),
        scratch_types=[
            pltpu.VMEM((CHUNKS_ST, C), jnp.int32),
            pltpu.VMEM((CHUNKS_ST, C), jnp.int32),
            pltpu.VMEM((C, H), jnp.float32),
            pltpu.VMEM_SHARED((ACC_ROWS, H), jnp.float32),
            pltpu.SemaphoreType.DMA,
        ],
    )


def kernel(x, edge_index, batch, W1, b1, g1, be1, W2, b2, g2, be2,
           W3, b3, g3, be3, MW1, Mb1, MW2, Mb2):
    pad = E_PAD - E
    pad_dst = N + (jnp.arange(pad, dtype=jnp.int32) % (ACC_ROWS - N))
    srcp = jnp.concatenate(
        [edge_index[0], jnp.zeros((pad,), jnp.int32)]).reshape(NW, CHUNKS, C)
    dstp = jnp.concatenate(
        [edge_index[1], pad_dst]).reshape(NW, CHUNKS, C)
    ext = CHUNKS_ST - CHUNKS
    srcp = jnp.concatenate([srcp, jnp.zeros((NW, ext, C), jnp.int32)], axis=1)
    dstp = jnp.concatenate(
        [dstp, jnp.full((NW, ext, C), DUMMY, jnp.int32)], axis=1)
    batch2 = batch.reshape(1, N)
    b1r, g1r, be1r = b1.reshape(1, H), g1.reshape(1, H), be1.reshape(1, H)
    b2r, g2r, be2r = b2.reshape(1, H), g2.reshape(1, H), be2.reshape(1, H)
    b3r, g3r, be3r = b3.reshape(1, H), g3.reshape(1, H), be3.reshape(1, H)
    mw1a, mw1b = MW1[:H], MW1[H:]
    mb1 = Mb1.reshape(1, H)
    mb2 = Mb2.reshape(1, 1)

    deg2 = _get_deg_kernel()(dstp)
    xw1 = pl.pallas_call(_xw_body, out_shape=_sds((N, H)))(x, W1)
    xs1, dinv = pl.pallas_call(
        _prep_body, out_shape=(_sds((N, H)), _sds((N, 1))))(deg2, xw1)

    p1 = _get_mp_kernel()(xs1, srcp, dstp)
    conv1 = pl.pallas_call(_conv_body, out_shape=_sds((N, H)))(
        p1, xs1, dinv, b1r)
    h1, xs2 = pl.pallas_call(
        _bn1_body, out_shape=(_sds((N, H)), _sds((N, H))))(
            conv1, g1r, be1r, dinv, W2)

    p2 = _get_mp_kernel()(xs2, srcp, dstp)
    conv2 = pl.pallas_call(_conv_body, out_shape=_sds((N, H)))(
        p2, xs2, dinv, b2r)
    h2, xs3 = pl.pallas_call(
        _bn2_body, out_shape=(_sds((N, H)), _sds((N, H))))(
            conv2, g2r, be2r, h1, dinv, W3)

    p3 = _get_mp_kernel()(xs3, srcp, dstp)
    conv3 = pl.pallas_call(_conv_body, out_shape=_sds((N, H)))(
        p3, xs3, dinv, b3r)
    h3 = pl.pallas_call(_bn3_body, out_shape=_sds((N, H)))(
        conv3, g3r, be3r, h2)
    out = pl.pallas_call(_final_body, out_shape=_sds((G, 1)))(
        h3, batch2, mw1a, mw1b, mb1, MW2, mb2)
    return out


# speedup vs baseline: 2.9068x; 2.9027x over previous
import functools

import jax
import jax.numpy as jnp
from jax import lax
from jax.experimental import pallas as pl
from jax.experimental.pallas import tpu as pltpu
from jax.experimental.pallas import tpu_sc as plsc

N = 10000
E = 320000
H = 128
G = 128
NC, NS, LANES = 2, 16, 16
NW = NC * NS
C = 128
CHUNKS = 79
CHUNKS_ST = CHUNKS + 1
C2 = 2 * C
M_ST = 24
M_STAGES = ((0, 0, 20), (16, 4, 20))
E_PAD = NW * CHUNKS * C
ACC_ROWS = 10240
ZROWS = ACC_ROWS // NS
DUMMY = N

@functools.cache
def _get_mesh():
    return plsc.VectorSubcoreMesh(
        core_axis_name="c", subcore_axis_name="s", num_cores=NC, num_subcores=NS)


_HIGH = lax.Precision.HIGHEST


def _fill(buf, rows, cols, val):
    @pl.loop(0, rows)
    def _(r):
        @pl.loop(0, cols, step=LANES)
        def _(cc):
            buf[r, pl.ds(cc, LANES)] = jnp.full((LANES,), val, buf.dtype)


def _deg_body(dst_hbm, out_hbm, didx, buf, acc, sem):
    cid = lax.axis_index("c")
    sid = lax.axis_index("s")
    wid = cid * NS + sid
    pltpu.sync_copy(dst_hbm.at[wid], didx)
    _fill(buf, C, LANES, 0.0)
    base = sid * ZROWS
    for k in range(ZROWS // C):
        pltpu.sync_copy(buf, acc.at[pl.ds(base + k * C, C)])
    _fill(buf, C, LANES, 1.0)
    plsc.subcore_barrier()

    @pl.loop(0, CHUNKS)
    def _(j):
        pltpu.sync_copy(buf, acc.at[didx.at[j]], add=True)

    plsc.subcore_barrier()
    pltpu.sync_copy(acc.at[pl.ds(base, ZROWS)],
                    out_hbm.at[cid, pl.ds(base, ZROWS)])


def _mp_body(xs_hbm, src_hbm, dst_hbm, out_hbm, sidx, didx, gbuf_a, acc,
             sem_a):
    cid = lax.axis_index("c")
    sid = lax.axis_index("s")
    wid = cid * NS + sid
    _fill(gbuf_a, C, H, 0.0)
    base = sid * ZROWS
    for k in range(ZROWS // C):
        pltpu.sync_copy(gbuf_a, acc.at[pl.ds(base + k * C, C)])
    plsc.subcore_barrier()

    pltpu.sync_copy(src_hbm.at[wid], sidx)
    pltpu.sync_copy(dst_hbm.at[wid], didx)

    @pl.loop(0, CHUNKS)
    def _(j):
        pltpu.async_copy(xs_hbm.at[sidx.at[j]], gbuf_a, sem_a).wait()
        pltpu.sync_copy(gbuf_a, acc.at[didx.at[j]], add=True)

    plsc.subcore_barrier()
    pltpu.sync_copy(acc.at[pl.ds(base, ZROWS)],
                    out_hbm.at[cid, pl.ds(base, ZROWS)])


def _xw_body(x_ref, w_ref, xw_ref):
    xw_ref[...] = jnp.dot(x_ref[...], w_ref[...],
                          preferred_element_type=jnp.float32, precision=_HIGH)


def _prep_body(deg2_ref, xw_ref, xs_ref, dinv_ref):
    d0 = deg2_ref[0]
    d1 = deg2_ref[1]
    deg = d0[:N, 0:1] + d1[:N, 0:1] + 1.0
    dinv = lax.rsqrt(deg)
    dinv_ref[...] = dinv
    xs_ref[...] = xw_ref[...] * dinv


def _bn_relu(conv, g, be):
    m = jnp.mean(conv, axis=0, keepdims=True)
    cc = conv - m
    v = jnp.mean(cc * cc, axis=0, keepdims=True)
    return jnp.maximum(cc * lax.rsqrt(v + 1e-5) * g + be, 0.0)


def _conv_body(p_ref, xs_ref, dinv_ref, b_ref, conv_ref):
    conv_ref[...] = ((p_ref[0][:N] + p_ref[1][:N] + xs_ref[...])
                     * dinv_ref[...] + b_ref[...])


def _bn1_body(conv_ref, g_ref, be_ref, dinv_ref, w_ref, h_ref, xsn_ref):
    h = _bn_relu(conv_ref[...], g_ref[...], be_ref[...])
    h_ref[...] = h
    xsn_ref[...] = jnp.dot(h, w_ref[...], preferred_element_type=jnp.float32,
                           precision=_HIGH) * dinv_ref[...]


def _bn2_body(conv_ref, g_ref, be_ref, res_ref, dinv_ref, w_ref,
              h_ref, xsn_ref):
    h = _bn_relu(conv_ref[...], g_ref[...], be_ref[...]) + res_ref[...]
    h_ref[...] = h
    xsn_ref[...] = jnp.dot(h, w_ref[...], preferred_element_type=jnp.float32,
                           precision=_HIGH) * dinv_ref[...]


def _bn3_body(conv_ref, g_ref, be_ref, res_ref, h_ref):
    h_ref[...] = _bn_relu(conv_ref[...], g_ref[...], be_ref[...]) + res_ref[...]


def _final_body(h_ref, batch_ref, mw1a_ref, mw1b_ref, mb1_ref, mw2_ref,
                mb2_ref, out_ref):
    h = h_ref[...]
    gids = lax.broadcasted_iota(jnp.int32, (G, N), 0)
    mask = (gids == batch_ref[...]).astype(jnp.float32)
    xs_pool = jnp.dot(mask, h, preferred_element_type=jnp.float32,
                      precision=_HIGH)
    cnt = jnp.sum(mask, axis=1, keepdims=True)
    xm = xs_pool / jnp.maximum(cnt, 1.0)
    z = jnp.dot(xs_pool, mw1a_ref[...], preferred_element_type=jnp.float32,
                precision=_HIGH)
    z = z + jnp.dot(xm, mw1b_ref[...], preferred_element_type=jnp.float32,
                    precision=_HIGH)
    z = jnp.maximum(z + mb1_ref[...], 0.0)
    out_ref[...] = jnp.dot(z, mw2_ref[...], preferred_element_type=jnp.float32,
                           precision=_HIGH) + mb2_ref[...]


def _sds(shape):
    return jax.ShapeDtypeStruct(shape, jnp.float32)


@functools.cache
def _get_deg_kernel():
    return pl.kernel(
        _deg_body,
        out_type=jax.ShapeDtypeStruct((NC, ACC_ROWS, LANES), jnp.float32),
        mesh=_get_mesh(),
        scratch_types=[
            pltpu.VMEM((CHUNKS_ST, C), jnp.int32),
            pltpu.VMEM((C, LANES), jnp.float32),
            pltpu.VMEM_SHARED((ACC_ROWS, LANES), jnp.float32),
            pltpu.SemaphoreType.DMA,
        ],
    )


@functools.cache
def _get_mp_kernel():
    return pl.kernel(
        _mp_body,
        out_type=jax.ShapeDtypeStruct((NC, ACC_ROWS, H), jnp.float32),
        mesh=_get_mesh(),
        scratch_types=[
            pltpu.VMEM((CHUNKS_ST, C), jnp.int32),
            pltpu.VMEM((CHUNKS_ST, C), jnp.int32),
            pltpu.VMEM((C, H), jnp.float32),
            pltpu.VMEM_SHARED((ACC_ROWS, H), jnp.float32),
            pltpu.SemaphoreType.DMA,
        ],
    )


def kernel(x, edge_index, batch, W1, b1, g1, be1, W2, b2, g2, be2,
           W3, b3, g3, be3, MW1, Mb1, MW2, Mb2):
    pad = E_PAD - E
    pad_i = jnp.arange(pad, dtype=jnp.int32)
    pad_src = pad_i % N
    pad_dst = N + (pad_i % (ACC_ROWS - N))
    srcp = jnp.concatenate(
        [edge_index[0], pad_src]).reshape(NW, CHUNKS, C)
    dstp = jnp.concatenate(
        [edge_index[1], pad_dst]).reshape(NW, CHUNKS, C)
    ext = CHUNKS_ST - CHUNKS
    srcp = jnp.concatenate([srcp, jnp.zeros((NW, ext, C), jnp.int32)], axis=1)
    dstp = jnp.concatenate(
        [dstp, jnp.full((NW, ext, C), DUMMY, jnp.int32)], axis=1)
    batch2 = batch.reshape(1, N)
    b1r, g1r, be1r = b1.reshape(1, H), g1.reshape(1, H), be1.reshape(1, H)
    b2r, g2r, be2r = b2.reshape(1, H), g2.reshape(1, H), be2.reshape(1, H)
    b3r, g3r, be3r = b3.reshape(1, H), g3.reshape(1, H), be3.reshape(1, H)
    mw1a, mw1b = MW1[:H], MW1[H:]
    mb1 = Mb1.reshape(1, H)
    mb2 = Mb2.reshape(1, 1)

    deg2 = _get_deg_kernel()(dstp)
    xw1 = pl.pallas_call(_xw_body, out_shape=_sds((N, H)))(x, W1)
    xs1, dinv = pl.pallas_call(
        _prep_body, out_shape=(_sds((N, H)), _sds((N, 1))))(deg2, xw1)

    p1 = _get_mp_kernel()(xs1, srcp, dstp)
    conv1 = pl.pallas_call(_conv_body, out_shape=_sds((N, H)))(
        p1, xs1, dinv, b1r)
    h1, xs2 = pl.pallas_call(
        _bn1_body, out_shape=(_sds((N, H)), _sds((N, H))))(
            conv1, g1r, be1r, dinv, W2)

    p2 = _get_mp_kernel()(xs2, srcp, dstp)
    conv2 = pl.pallas_call(_conv_body, out_shape=_sds((N, H)))(
        p2, xs2, dinv, b2r)
    h2, xs3 = pl.pallas_call(
        _bn2_body, out_shape=(_sds((N, H)), _sds((N, H))))(
            conv2, g2r, be2r, h1, dinv, W3)

    p3 = _get_mp_kernel()(xs3, srcp, dstp)
    conv3 = pl.pallas_call(_conv_body, out_shape=_sds((N, H)))(
        p3, xs3, dinv, b3r)
    h3 = pl.pallas_call(_bn3_body, out_shape=_sds((N, H)))(
        conv3, g3r, be3r, h2)
    out = pl.pallas_call(_final_body, out_shape=_sds((G, 1)))(
        h3, batch2, mw1a, mw1b, mb1, MW2, mb2)
    return out
